# baseline (speedup 1.0000x reference)
import functools

import jax
import jax.numpy as jnp
from jax import lax
from jax.experimental import pallas as pl
from jax.experimental.pallas import tpu as pltpu
from jax.experimental.pallas import tpu_sc as plsc

N = 1000000
N_D = 32
BATCH = 16384

_info = plsc.get_sparse_core_info()
_NC, _NS = _info.num_cores, _info.num_subcores
_NW = _NC * _NS
_BPW = BATCH // _NW


def _gather_body(idx_hbm, tab_hbm, out_hbm, idx_v, q_v, rows_v, sem_i, sem_g):
    wid = lax.axis_index("s") * _NC + lax.axis_index("c")
    base = wid * _BPW
    pltpu.async_copy(idx_hbm.at[pl.ds(base, _BPW)], idx_v, sem_i).wait()

    def fill(b, carry):
        j0 = b * 16
        q_v[pl.ds(j0, 16)] = idx_v[pl.ds(j0, 16)] >> 1
        return carry

    lax.fori_loop(0, _BPW // 16, fill, 0)
    pltpu.async_copy(tab_hbm.at[q_v], rows_v, sem_g).wait()
    pltpu.async_copy(rows_v, out_hbm.at[pl.ds(base, _BPW), :], sem_g).wait()


@jax.jit
def kernel(idx, Wa, Ws):
    mesh = plsc.VectorSubcoreMesh(core_axis_name="c", subcore_axis_name="s")
    run = functools.partial(
        pl.kernel,
        mesh=mesh,
        out_type=jax.ShapeDtypeStruct((BATCH, 128), jnp.float32),
        scratch_types=[
            pltpu.VMEM((_BPW,), jnp.int32),
            pltpu.VMEM((_BPW,), jnp.int32),
            pltpu.VMEM((_BPW, 128), jnp.float32),
            pltpu.SemaphoreType.DMA,
            pltpu.SemaphoreType.DMA,
        ],
    )(_gather_body)
    Wa2 = Wa.reshape(N // 2, 2 * N_D)
    Ws2 = Ws.reshape(N // 2, 2 * N_D)
    tab = jnp.concatenate(
        [Wa2[:, :N_D], Ws2[:, :N_D], Wa2[:, N_D:], Ws2[:, N_D:]], axis=1)
    out = run(idx, tab)
    even = ((idx & 1) == 0)[:, None]
    za = jnp.where(even, out[:, :N_D], out[:, 2 * N_D:3 * N_D])
    zs = jnp.where(even, out[:, N_D:2 * N_D], out[:, 3 * N_D:])
    return (za, zs)
